# speedup vs baseline: 5.8616x; 5.8616x over previous
import jax
import jax.numpy as jnp
from jax import lax
from jax.experimental import pallas as pl
from jax.experimental.pallas import tpu as pltpu
from jax.experimental.pallas import tpu_sc as plsc

S = 16384
K = 32
D = 128
L = 16
NC, NS = 2, 16
NW = NC * NS
S_W = S // NW
C = 8
NCHUNK = S_W // C


def _sc_kernel(x_hbm, idx_hbm, out_hbm, idx_v, rows_v, out_v, sem):
    wid = lax.axis_index("s") * NC + lax.axis_index("c")
    seg0 = wid * S_W

    @pl.loop(0, NCHUNK)
    def _chunk(g):
        seg = seg0 + g * C
        pltpu.sync_copy(idx_hbm.at[pl.ds(seg * K, C * K)], idx_v)
        pltpu.async_copy(x_hbm.at[idx_v], rows_v, sem).wait()

        @pl.loop(0, C)
        def _seg(s):
            row0 = s * K
            for j in range(D // L):
                cols = pl.ds(j * L, L)
                vals = [rows_v[row0 + k, cols] for k in range(K)]
                while len(vals) > 1:
                    vals = [vals[i] + vals[i + 1] for i in range(0, len(vals), 2)]
                out_v[s, cols] = vals[0] * (1.0 / K)

        pltpu.sync_copy(out_v, out_hbm.at[pl.ds(seg, C)])


@jax.jit
def kernel(X, GP_info):
    idx = GP_info.reshape(-1).astype(jnp.int32)
    mesh = plsc.VectorSubcoreMesh(
        core_axis_name="c", subcore_axis_name="s", num_cores=NC, num_subcores=NS
    )
    return pl.kernel(
        _sc_kernel,
        out_type=jax.ShapeDtypeStruct((S, D), jnp.float32),
        mesh=mesh,
        scratch_types=[
            pltpu.VMEM((C * K,), jnp.int32),
            pltpu.VMEM((C * K, D), jnp.float32),
            pltpu.VMEM((C, D), jnp.float32),
            pltpu.SemaphoreType.DMA,
        ],
    )(X, idx)


# speedup vs baseline: 9.8639x; 1.6828x over previous
import jax
import jax.numpy as jnp
from jax import lax
from jax.experimental import pallas as pl
from jax.experimental.pallas import tpu as pltpu
from jax.experimental.pallas import tpu_sc as plsc

S = 16384
K = 32
D = 128
L = 16
NC, NS = 2, 16
NW = NC * NS
S_W = S // NW
C = 8
NCHUNK = S_W // C


def _sc_kernel(x_hbm, idx_hbm, out_hbm, idx_v0, idx_v1, rows_v0, rows_v1,
               out_v0, out_v1, gsem0, gsem1, osem0, osem1):
    idx_v = (idx_v0, idx_v1)
    rows_v = (rows_v0, rows_v1)
    out_v = (out_v0, out_v1)
    gsems = (gsem0, gsem1)
    osems = (osem0, osem1)
    wid = lax.axis_index("s") * NC + lax.axis_index("c")
    seg0 = wid * S_W

    def start_gather(g, b):
        pltpu.sync_copy(idx_hbm.at[pl.ds((seg0 + g * C) * K, C * K)], idx_v[b])
        pltpu.async_copy(x_hbm.at[idx_v[b]], rows_v[b], gsems[b])

    def out_copy_desc(g, b):
        return pltpu.make_async_copy(
            out_v[b], out_hbm.at[pl.ds(seg0 + g * C, C)], osems[b])

    start_gather(0, 0)

    @pl.loop(0, NCHUNK, step=2)
    def _chunk(g):
        for b in range(2):
            cur = g + b
            nb = (b + 1) % 2

            @pl.when(cur + 1 < NCHUNK)
            def _():
                start_gather(cur + 1, nb)

            pltpu.make_async_copy(x_hbm.at[idx_v[b]], rows_v[b],
                                  gsems[b]).wait()

            @pl.when(cur >= 2)
            def _():
                out_copy_desc(cur - 2, b).wait()

            rv = rows_v[b]
            ov = out_v[b]

            @pl.loop(0, C)
            def _seg(s):
                row0 = s * K
                for j in range(D // L):
                    cols = pl.ds(j * L, L)
                    vals = [rv[row0 + k, cols] for k in range(K)]
                    while len(vals) > 1:
                        vals = [vals[i] + vals[i + 1]
                                for i in range(0, len(vals), 2)]
                    ov[s, cols] = vals[0] * (1.0 / K)

            out_copy_desc(cur, b).start()

    out_copy_desc(NCHUNK - 2, 0).wait()
    out_copy_desc(NCHUNK - 1, 1).wait()


@jax.jit
def kernel(X, GP_info):
    idx = GP_info.reshape(-1).astype(jnp.int32)
    mesh = plsc.VectorSubcoreMesh(
        core_axis_name="c", subcore_axis_name="s", num_cores=NC,
        num_subcores=NS
    )
    return pl.kernel(
        _sc_kernel,
        out_type=jax.ShapeDtypeStruct((S, D), jnp.float32),
        mesh=mesh,
        scratch_types=[
            pltpu.VMEM((C * K,), jnp.int32),
            pltpu.VMEM((C * K,), jnp.int32),
            pltpu.VMEM((C * K, D), jnp.float32),
            pltpu.VMEM((C * K, D), jnp.float32),
            pltpu.VMEM((C, D), jnp.float32),
            pltpu.VMEM((C, D), jnp.float32),
            pltpu.SemaphoreType.DMA,
            pltpu.SemaphoreType.DMA,
            pltpu.SemaphoreType.DMA,
            pltpu.SemaphoreType.DMA,
        ],
    )(X, idx)


# speedup vs baseline: 11.4796x; 1.1638x over previous
import jax
import jax.numpy as jnp
from jax import lax
from jax.experimental import pallas as pl
from jax.experimental.pallas import tpu as pltpu
from jax.experimental.pallas import tpu_sc as plsc

S = 16384
K = 32
D = 128
L = 16
NC, NS = 2, 16
NW = NC * NS
S_W = S // NW
C = 8
NCHUNK = S_W // C


def _sc_kernel(x_hbm, idx_hbm, out_hbm, idx_v0, idx_v1, rows_v0, rows_v1,
               out_v0, out_v1, isem0, isem1, gsem0, gsem1, osem0, osem1):
    idx_v = (idx_v0, idx_v1)
    rows_v = (rows_v0, rows_v1)
    out_v = (out_v0, out_v1)
    isems = (isem0, isem1)
    gsems = (gsem0, gsem1)
    osems = (osem0, osem1)
    wid = lax.axis_index("s") * NC + lax.axis_index("c")
    seg0 = wid * S_W

    def idx_copy_desc(g, b):
        return pltpu.make_async_copy(
            idx_hbm.at[pl.ds((seg0 + g * C) * K, C * K)], idx_v[b], isems[b])

    def gather_desc(b):
        return pltpu.make_async_copy(x_hbm.at[idx_v[b]], rows_v[b], gsems[b])

    def out_copy_desc(g, b):
        return pltpu.make_async_copy(
            out_v[b], out_hbm.at[pl.ds(seg0 + g * C, C)], osems[b])

    idx_copy_desc(0, 0).start()
    idx_copy_desc(0, 0).wait()
    gather_desc(0).start()
    idx_copy_desc(1, 1).start()

    @pl.loop(0, NCHUNK, step=2)
    def _chunk(g):
        for b in range(2):
            cur = g + b
            nb = (b + 1) % 2

            @pl.when(cur + 1 < NCHUNK)
            def _():
                idx_copy_desc(cur + 1, nb).wait()
                gather_desc(nb).start()

            gather_desc(b).wait()

            @pl.when(cur + 2 < NCHUNK)
            def _():
                idx_copy_desc(cur + 2, b).start()

            @pl.when(cur >= 2)
            def _():
                out_copy_desc(cur - 2, b).wait()

            rv = rows_v[b]
            ov = out_v[b]

            @pl.loop(0, C)
            def _seg(s):
                row0 = s * K
                for j in range(D // L):
                    cols = pl.ds(j * L, L)
                    vals = [rv[row0 + k, cols] for k in range(K)]
                    while len(vals) > 1:
                        vals = [vals[i] + vals[i + 1]
                                for i in range(0, len(vals), 2)]
                    ov[s, cols] = vals[0] * (1.0 / K)

            out_copy_desc(cur, b).start()

    out_copy_desc(NCHUNK - 2, 0).wait()
    out_copy_desc(NCHUNK - 1, 1).wait()


@jax.jit
def kernel(X, GP_info):
    idx = GP_info.reshape(-1).astype(jnp.int32)
    mesh = plsc.VectorSubcoreMesh(
        core_axis_name="c", subcore_axis_name="s", num_cores=NC,
        num_subcores=NS
    )
    return pl.kernel(
        _sc_kernel,
        out_type=jax.ShapeDtypeStruct((S, D), jnp.float32),
        mesh=mesh,
        scratch_types=[
            pltpu.VMEM((C * K,), jnp.int32),
            pltpu.VMEM((C * K,), jnp.int32),
            pltpu.VMEM((C * K, D), jnp.float32),
            pltpu.VMEM((C * K, D), jnp.float32),
            pltpu.VMEM((C, D), jnp.float32),
            pltpu.VMEM((C, D), jnp.float32),
            pltpu.SemaphoreType.DMA,
            pltpu.SemaphoreType.DMA,
            pltpu.SemaphoreType.DMA,
            pltpu.SemaphoreType.DMA,
            pltpu.SemaphoreType.DMA,
            pltpu.SemaphoreType.DMA,
        ],
    )(X, idx)
